# baseline (speedup 1.0000x reference)
import functools

import jax
import jax.numpy as jnp
from jax import lax
from jax.experimental import pallas as pl
from jax.experimental.pallas import tpu as pltpu
from jax.experimental.pallas import tpu_sc as plsc

N = 10000
E = 160000
D = 128
R2 = 200
NB = 4
CPAD = 64

NC, NS, L = 2, 16, 16
NW = NC * NS
NP = 10240
NACC = 10240
HCR = NACC // D

_mesh = plsc.VectorSubcoreMesh(
    core_axis_name="c", subcore_axis_name="s", num_cores=NC, num_subcores=NS)
_sc_params = pltpu.CompilerParams(needs_layout_passes=False)

ROWS_A = NP // NW

KE = 32
E_PAD = 163840
NCH_P = E_PAD // (NW * KE)
KA = 128
NCH_A = E_PAD // (NW * KA)


@functools.partial(
    pl.kernel,
    out_type=(jax.ShapeDtypeStruct((NP, D), jnp.float32),
              jax.ShapeDtypeStruct((NP,), jnp.int32),
              jax.ShapeDtypeStruct((2 * HCR, D), jnp.float32),
              jax.ShapeDtypeStruct((E_PAD * NB,), jnp.float32),
              jax.ShapeDtypeStruct((E_PAD * NB,), jnp.float32)),
    mesh=_mesh,
    scratch_types=[
        pltpu.VMEM((ROWS_A,), jnp.int32),
        pltpu.VMEM((ROWS_A, D), jnp.float32),
        pltpu.VMEM((NP,), jnp.int32),
        pltpu.VMEM((ROWS_A,), jnp.int32),
        pltpu.VMEM((KA,), jnp.int32),
        pltpu.VMEM((KA,), jnp.int32),
        pltpu.VMEM((KA,), jnp.float32),
        pltpu.VMEM((R2 * NB,), jnp.float32),
        pltpu.VMEM((R2 * NB,), jnp.float32),
        pltpu.VMEM((KA * NB,), jnp.float32),
        pltpu.VMEM((KA * NB,), jnp.float32),
        pltpu.VMEM((HCR, D), jnp.float32),
        pltpu.VMEM((HCR,), jnp.int32),
        pltpu.VMEM_SHARED((HCR, D), jnp.float32),
        pltpu.SemaphoreType.DMA,
    ],
    compiler_params=_sc_params,
)
def _sc_gather(ent_hbm, emb_hbm, e2c_hbm, dst_hbm, et_hbm, nf_hbm,
               att1_hbm, att2_hbm,
               x0_hbm, g_hbm, pc_hbm, cf1_hbm, cf2_hbm,
               ent_v, rows_v, e2c_v, g_v, dstv, etv, nfv,
               att1_v, att2_v, cf1_v, cf2_v, hist_v, rowidx_v, cnt_sh,
               sem):
    cid = lax.axis_index("c")
    sid = lax.axis_index("s")
    wid = sid * NC + cid
    base = wid * ROWS_A
    pltpu.sync_copy(ent_hbm.at[pl.ds(base, ROWS_A)], ent_v)
    pltpu.async_copy(emb_hbm.at[ent_v], rows_v, sem).wait()
    pltpu.sync_copy(rows_v, x0_hbm.at[pl.ds(base, ROWS_A)])
    pltpu.sync_copy(e2c_hbm, e2c_v)

    def body(k, carry):
        idx = ent_v[pl.ds(k * L, L)]
        g_v[pl.ds(k * L, L)] = plsc.load_gather(e2c_v, [idx])
        return carry

    lax.fori_loop(0, ROWS_A // L, body, 0)
    pltpu.sync_copy(g_v, g_hbm.at[pl.ds(base, ROWS_A)])

    zv = jnp.zeros((L,), jnp.float32)
    lane = lax.broadcasted_iota(jnp.int32, (L,), 0)

    def zrow(i, carry):
        for j in range(D // L):
            hist_v[i, pl.ds(j * L, L)] = zv
        return carry

    lax.fori_loop(0, HCR, zrow, 0)
    for k in range(HCR // L):
        rowidx_v[pl.ds(k * L, L)] = lane + (k * L)
    ntc = HCR // 8

    @pl.when(sid < ntc)
    def _zero_cnt():
        pltpu.sync_copy(hist_v.at[pl.ds(0, 8)],
                        cnt_sh.at[pl.ds(sid * 8, 8)])

    plsc.subcore_barrier()
    pltpu.sync_copy(att1_hbm, att1_v)
    pltpu.sync_copy(att2_hbm, att2_v)

    ones_v = jnp.ones((L,), jnp.float32)
    masks = [lane == j for j in range(L)]

    def chunk(c, carry):
        off = (c * NW + wid) * KA
        pltpu.sync_copy(dst_hbm.at[pl.ds(off, KA)], dstv)
        pltpu.sync_copy(et_hbm.at[pl.ds(off, KA)], etv)
        pltpu.sync_copy(nf_hbm.at[pl.ds(off, KA)], nfv)
        for k in range(KA // L):
            dv = dstv[pl.ds(k * L, L)]
            r = lax.shift_right_logical(dv, 7)
            q = lax.bitwise_and(dv, D - 1)
            for j in range(L):
                plsc.addupdate_scatter(hist_v, [r, q], ones_v, mask=masks[j])
            tv = etv[pl.ds(k * L, L)] * NB
            nv = nfv[pl.ds(k * L, L)]
            cidx = (lane + k * L) * NB
            for b in range(NB):
                a1 = plsc.load_gather(att1_v, [tv + b])
                plsc.store_scatter(cf1_v, [cidx + b], a1 * nv)
                a2 = plsc.load_gather(att2_v, [tv + b])
                plsc.store_scatter(cf2_v, [cidx + b], a2 * nv)
        pltpu.sync_copy(cf1_v, cf1_hbm.at[pl.ds(off * NB, KA * NB)])
        pltpu.sync_copy(cf2_v, cf2_hbm.at[pl.ds(off * NB, KA * NB)])
        return carry

    lax.fori_loop(0, NCH_A, chunk, 0)
    pltpu.sync_copy(hist_v, cnt_sh.at[rowidx_v], add=True)
    plsc.subcore_barrier()

    @pl.when(sid < ntc)
    def _write_cnt():
        pltpu.sync_copy(cnt_sh.at[pl.ds(sid * 8, 8)],
                        hist_v.at[pl.ds(0, 8)])
        pltpu.sync_copy(hist_v.at[pl.ds(0, 8)],
                        pc_hbm.at[pl.ds(cid * HCR + sid * 8, 8)])


STRIPE = NACC // NS


@functools.partial(
    pl.kernel,
    out_type=jax.ShapeDtypeStruct((2 * NACC, D), jnp.float32),
    mesh=_mesh,
    scratch_types=[
        pltpu.VMEM((KE,), jnp.int32),
        pltpu.VMEM((KE,), jnp.int32),
        pltpu.VMEM((KE,), jnp.int32),
        pltpu.VMEM((KE,), jnp.int32),
        pltpu.VMEM((KE * NB + L,), jnp.float32),
        pltpu.VMEM((KE * NB + L,), jnp.float32),
        pltpu.VMEM((KE, NB * D), jnp.float32),
        pltpu.VMEM((KE, NB * D), jnp.float32),
        pltpu.VMEM((KE, D), jnp.float32),
        pltpu.VMEM((KE, D), jnp.float32),
        pltpu.VMEM((KE,), jnp.int32),
        pltpu.VMEM((KE,), jnp.int32),
        pltpu.VMEM_SHARED((NACC, D), jnp.float32),
        pltpu.SemaphoreType.DMA,
        pltpu.SemaphoreType.DMA,
        pltpu.SemaphoreType.DMA,
        pltpu.SemaphoreType.DMA,
        pltpu.SemaphoreType.DMA,
        pltpu.SemaphoreType.DMA,
        pltpu.SemaphoreType.DMA,
        pltpu.SemaphoreType.DMA,
        pltpu.SemaphoreType.DMA,
        pltpu.SemaphoreType.DMA,
    ],
    compiler_params=_sc_params,
)
def _sc_edge(yb_hbm, src_hbm, dst_hbm, cf_hbm, z_hbm, p_hbm,
             srcv0, srcv1, dstv0, dstv1, cfv0, cfv1, rows0, rows1,
             msg0, msg1, dstm0, dstm1, acc_sh,
             ss0, ss1, sd0, sd1, sc0, sc1, sg0, sg1, sa0, sa1):
    cid = lax.axis_index("c")
    sid = lax.axis_index("s")
    wid = sid * NC + cid
    srcv = [srcv0, srcv1]
    dstv = [dstv0, dstv1]
    cfv = [cfv0, cfv1]
    rows = [rows0, rows1]
    msg = [msg0, msg1]
    dstm = [dstm0, dstm1]
    sem_s = [ss0, ss1]
    sem_d = [sd0, sd1]
    sem_c = [sc0, sc1]
    sem_g = [sg0, sg1]
    sem_a = [sa0, sa1]

    pltpu.sync_copy(z_hbm.at[pl.ds(sid * STRIPE, STRIPE)],
                    acc_sh.at[pl.ds(sid * STRIPE, STRIPE)])
    plsc.subcore_barrier()

    def issue_idx(g, s):
        off = (g * NW + wid) * KE
        pltpu.make_async_copy(src_hbm.at[pl.ds(off, KE)],
                              srcv[s], sem_s[s]).start()
        pltpu.make_async_copy(dst_hbm.at[pl.ds(off, KE)],
                              dstv[s], sem_d[s]).start()
        pltpu.make_async_copy(cf_hbm.at[pl.ds(off * NB, KE * NB)],
                              cfv[s].at[pl.ds(0, KE * NB)], sem_c[s]).start()

    def wait_idx(s):
        pltpu.make_async_copy(src_hbm.at[pl.ds(0, KE)],
                              srcv[s], sem_s[s]).wait()
        pltpu.make_async_copy(dst_hbm.at[pl.ds(0, KE)],
                              dstv[s], sem_d[s]).wait()
        pltpu.make_async_copy(cf_hbm.at[pl.ds(0, KE * NB)],
                              cfv[s].at[pl.ds(0, KE * NB)], sem_c[s]).wait()

    def start_gather(s):
        pltpu.make_async_copy(yb_hbm.at[srcv[s]], rows[s], sem_g[s]).start()

    def wait_gather(s):
        pltpu.make_async_copy(yb_hbm.at[srcv[s]], rows[s], sem_g[s]).wait()

    issue_idx(0, 0)
    wait_idx(0)
    start_gather(0)
    issue_idx(1, 1)

    def wait_scatter(s):
        pltpu.make_async_copy(msg[s], acc_sh.at[dstm[s]], sem_a[s]).wait()

    def half(g, cur, nxt):
        @pl.when(g + 1 < NCH_P)
        def _advance():
            wait_idx(nxt)
            start_gather(nxt)

        wait_gather(cur)

        @pl.when(g >= 2)
        def _drain():
            wait_scatter(cur)

        rv = rows[cur]
        cv_ref = cfv[cur]
        mv = msg[cur]

        def one_edge(e):
            cv = cv_ref[pl.ds(e * NB, L)]
            c0 = cv[0]
            c1 = cv[1]
            c2 = cv[2]
            c3 = cv[3]
            for j in range(D // L):
                v = rv[e, pl.ds(j * L, L)] * c0
                v = v + rv[e, pl.ds(D + j * L, L)] * c1
                v = v + rv[e, pl.ds(2 * D + j * L, L)] * c2
                v = v + rv[e, pl.ds(3 * D + j * L, L)] * c3
                mv[e, pl.ds(j * L, L)] = v

        def edge2(i, ecarry):
            one_edge(2 * i)
            one_edge(2 * i + 1)
            return ecarry

        lax.fori_loop(0, KE // 2, edge2, 0)
        for k in range(KE // L):
            dstm[cur][pl.ds(k * L, L)] = dstv[cur][pl.ds(k * L, L)]
        pltpu.async_copy(mv, acc_sh.at[dstm[cur]], sem_a[cur], add=True)

        @pl.when(g + 2 < NCH_P)
        def _prefetch():
            issue_idx(g + 2, cur)

    def pair(p, carry):
        half(2 * p, 0, 1)
        half(2 * p + 1, 1, 0)
        return carry

    lax.fori_loop(0, NCH_P // 2, pair, 0)
    wait_scatter(0)
    wait_scatter(1)
    plsc.subcore_barrier()
    pltpu.sync_copy(acc_sh.at[pl.ds(sid * STRIPE, STRIPE)],
                    p_hbm.at[pl.ds(cid * NACC + sid * STRIPE, STRIPE)])


BM = 256
BN = 400
G = N // BN


def _mm_body(x_ref, w_ref, yb_ref, yr_ref):
    y = jnp.dot(x_ref[...], w_ref[...], preferred_element_type=jnp.float32)
    yb_ref[...] = y[:, :NB * D]
    yr_ref[...] = y[:, NB * D:]


_mm = pl.pallas_call(
    _mm_body,
    grid=(NP // BM,),
    in_specs=[
        pl.BlockSpec((BM, D), lambda i: (i, 0)),
        pl.BlockSpec((D, (NB + 1) * D), lambda i: (0, 0)),
    ],
    out_specs=[
        pl.BlockSpec((BM, NB * D), lambda i: (i, 0)),
        pl.BlockSpec((BM, D), lambda i: (i, 0)),
    ],
    out_shape=[
        jax.ShapeDtypeStruct((NP, NB * D), jnp.float32),
        jax.ShapeDtypeStruct((NP, D), jnp.float32),
    ],
)


BE = 1024
PCB = BE // D


def _epi_body(p0_ref, p1_ref, pc0_ref, pc1_ref, yr_ref, b_ref, out_ref,
              *, relu):
    s = p0_ref[...] + p1_ref[...]
    pc = pc0_ref[...] + pc1_ref[...]
    oh = (lax.shift_right_logical(
              lax.broadcasted_iota(jnp.int32, (BE, PCB), 0), 7)
          == lax.broadcasted_iota(jnp.int32, (BE, PCB), 1)).astype(
              jnp.float32)
    psel = jnp.dot(oh, pc, preferred_element_type=jnp.float32)
    ri = lax.broadcasted_iota(jnp.int32, (BE, D), 0)
    li = lax.broadcasted_iota(jnp.int32, (BE, D), 1)
    cnt = jnp.sum(jnp.where(li == lax.bitwise_and(ri, D - 1), psel, 0.0),
                  axis=1, keepdims=True)
    o = s / jnp.maximum(cnt, 1.0) + yr_ref[...] + b_ref[...]
    if relu:
        o = jnp.maximum(o, 0.0)
    out_ref[...] = o


def _make_epi(relu):
    return pl.pallas_call(
        functools.partial(_epi_body, relu=relu),
        grid=(NP // BE,),
        in_specs=[
            pl.BlockSpec((BE, D), lambda i: (i, 0)),
            pl.BlockSpec((BE, D), lambda i: (NACC // BE + i, 0)),
            pl.BlockSpec((PCB, D), lambda i: (i, 0)),
            pl.BlockSpec((PCB, D), lambda i: (HCR // PCB + i, 0)),
            pl.BlockSpec((BE, D), lambda i: (i, 0)),
            pl.BlockSpec((D,), lambda i: (0,)),
        ],
        out_specs=pl.BlockSpec((BE, D), lambda i: (i, 0)),
        out_shape=jax.ShapeDtypeStruct((NP, D), jnp.float32),
    )


_epi_relu = _make_epi(True)
_epi_lin = _make_epi(False)


def _stats_body(x_ref, c_ref, cs_ref, cc_ref):
    @pl.when(pl.program_id(0) == 0)
    def _init():
        cs_ref[...] = jnp.zeros_like(cs_ref)
        cc_ref[...] = jnp.zeros_like(cc_ref)

    cls = c_ref[0, 0, :]
    oht = (lax.broadcasted_iota(jnp.int32, (CPAD, BN), 0)
           == cls[None, :]).astype(jnp.float32)
    cs_ref[...] += jnp.dot(oht, x_ref[...], preferred_element_type=jnp.float32)
    cc_ref[...] += jnp.broadcast_to(
        jnp.sum(oht, axis=1, keepdims=True), (CPAD, D))


_stats = pl.pallas_call(
    _stats_body,
    grid=(G,),
    in_specs=[
        pl.BlockSpec((BN, D), lambda i: (i, 0)),
        pl.BlockSpec((1, 1, BN), lambda i: (i, 0, 0)),
    ],
    out_specs=[
        pl.BlockSpec((CPAD, D), lambda i: (0, 0)),
        pl.BlockSpec((CPAD, D), lambda i: (0, 0)),
    ],
    out_shape=[
        jax.ShapeDtypeStruct((CPAD, D), jnp.float32),
        jax.ShapeDtypeStruct((CPAD, D), jnp.float32),
    ],
)


def _fin_body(e_ref, g_ref, cs_ref, cc_ref, w_ref, out_ref):
    fvs = jax.nn.sigmoid(cs_ref[...] / jnp.maximum(cc_ref[...], 1.0))
    gcls = g_ref[0, 0, :]
    oh = (gcls[:, None]
          == lax.broadcasted_iota(jnp.int32, (BN, CPAD), 1)).astype(jnp.float32)
    cf = jnp.dot(oh, fvs, preferred_element_type=jnp.float32)
    e2 = (jnp.dot(e_ref[...], w_ref[:D, :], preferred_element_type=jnp.float32)
          + jnp.dot(cf, w_ref[D:, :], preferred_element_type=jnp.float32))
    nrm = jnp.sqrt(jnp.sum(e2 * e2, axis=1, keepdims=True))
    out_ref[...] = e2 / jnp.maximum(nrm, 1e-12)


_fin = pl.pallas_call(
    _fin_body,
    grid=(G,),
    in_specs=[
        pl.BlockSpec((BN, D), lambda i: (i, 0)),
        pl.BlockSpec((1, 1, BN), lambda i: (i, 0, 0)),
        pl.BlockSpec((CPAD, D), lambda i: (0, 0)),
        pl.BlockSpec((CPAD, D), lambda i: (0, 0)),
        pl.BlockSpec((2 * D, D), lambda i: (0, 0)),
    ],
    out_specs=pl.BlockSpec((BN, D), lambda i: (i, 0)),
    out_shape=jax.ShapeDtypeStruct((N, D), jnp.float32),
)


def kernel(entity, edge_index, edge_type, edge_norm, emb_e,
           basis1, att1, root1, bias1, basis2, att2, root2, bias2,
           fc_w, entity2class):
    entity_p = jnp.concatenate(
        [entity, jnp.zeros((NP - N,), jnp.int32)])
    e2c_p = jnp.concatenate(
        [entity2class, jnp.zeros((NP - N,), jnp.int32)])
    npad = E_PAD - E
    src = jnp.concatenate([edge_index[0], jnp.zeros((npad,), jnp.int32)])
    dst = jnp.concatenate(
        [edge_index[1], jnp.full((npad,), NACC - 1, jnp.int32)])
    et_p = jnp.concatenate([edge_type, jnp.zeros((npad,), jnp.int32)])
    nf_p = jnp.concatenate([edge_norm, jnp.zeros((npad,), jnp.float32)])
    w1 = jnp.concatenate(
        [basis1[0], basis1[1], basis1[2], basis1[3], root1], axis=1)
    w2 = jnp.concatenate(
        [basis2[0], basis2[1], basis2[2], basis2[3], root2], axis=1)
    zeros_acc = jnp.zeros((NACC, D), jnp.float32)

    x0, g, pc, cf1, cf2 = _sc_gather(
        entity_p, emb_e, e2c_p, dst, et_p, nf_p,
        att1.reshape(-1), att2.reshape(-1))
    cs, cc = _stats(emb_e, entity2class.reshape(G, 1, BN))

    y1b, y1r = _mm(x0, w1)
    p1 = _sc_edge(y1b, src, dst, cf1, zeros_acc)
    e1 = _epi_relu(p1, p1, pc, pc, y1r, bias1)

    y2b, y2r = _mm(e1, w2)
    p2 = _sc_edge(y2b, src, dst, cf2, zeros_acc)
    e1f = _epi_lin(p2, p2, pc, pc, y2r, bias2)

    g3d = g[:N].reshape(G, 1, BN)
    return _fin(e1f, g3d, cs, cc, fc_w.T)


# speedup vs baseline: 1.2500x; 1.2500x over previous
import functools

import jax
import jax.numpy as jnp
import numpy as np
from jax import lax
from jax.experimental import pallas as pl
from jax.experimental.pallas import tpu as pltpu
from jax.experimental.pallas import tpu_sc as plsc

N = 10000
E = 160000
D = 128
R2 = 200
NB = 4
CPAD = 64

NC, NS, L = 2, 16, 16
NW = NC * NS
NP = 10240
NACC = 10240
HCR = NACC // D

_LO_IDX = np.array([32 * (k // 16) + k % 16 for k in range(256)])
_HI_IDX = _LO_IDX + 16

_mesh = plsc.VectorSubcoreMesh(
    core_axis_name="c", subcore_axis_name="s", num_cores=NC, num_subcores=NS)
_sc_params = pltpu.CompilerParams(needs_layout_passes=False)

ROWS_A = NP // NW

KE = 32
E_PAD = 163840
NCH_P = E_PAD // (NW * KE)
KA = 128
NCH_A = E_PAD // (NW * KA)


@functools.partial(
    pl.kernel,
    out_type=(jax.ShapeDtypeStruct((NP, D), jnp.float32),
              jax.ShapeDtypeStruct((NP,), jnp.int32),
              jax.ShapeDtypeStruct((2 * HCR, D), jnp.float32),
              jax.ShapeDtypeStruct((E_PAD * NB,), jnp.float32),
              jax.ShapeDtypeStruct((E_PAD * NB,), jnp.float32)),
    mesh=_mesh,
    scratch_types=[
        pltpu.VMEM((ROWS_A,), jnp.int32),
        pltpu.VMEM((ROWS_A, D), jnp.float32),
        pltpu.VMEM((NP,), jnp.int32),
        pltpu.VMEM((ROWS_A,), jnp.int32),
        pltpu.VMEM((KA,), jnp.int32),
        pltpu.VMEM((KA,), jnp.int32),
        pltpu.VMEM((KA,), jnp.float32),
        pltpu.VMEM((R2 * NB,), jnp.float32),
        pltpu.VMEM((R2 * NB,), jnp.float32),
        pltpu.VMEM((KA * NB,), jnp.float32),
        pltpu.VMEM((KA * NB,), jnp.float32),
        pltpu.VMEM((HCR, D), jnp.float32),
        pltpu.VMEM((HCR,), jnp.int32),
        pltpu.VMEM_SHARED((HCR, D), jnp.float32),
        pltpu.SemaphoreType.DMA,
    ],
    compiler_params=_sc_params,
)
def _sc_gather(ent_hbm, emb_hbm, e2c_hbm, dst_hbm, et_hbm, nf_hbm,
               att1_hbm, att2_hbm,
               x0_hbm, g_hbm, pc_hbm, cf1_hbm, cf2_hbm,
               ent_v, rows_v, e2c_v, g_v, dstv, etv, nfv,
               att1_v, att2_v, cf1_v, cf2_v, hist_v, rowidx_v, cnt_sh,
               sem):
    cid = lax.axis_index("c")
    sid = lax.axis_index("s")
    wid = sid * NC + cid
    base = wid * ROWS_A
    pltpu.sync_copy(ent_hbm.at[pl.ds(base, ROWS_A)], ent_v)
    pltpu.async_copy(emb_hbm.at[ent_v], rows_v, sem).wait()
    pltpu.sync_copy(rows_v, x0_hbm.at[pl.ds(base, ROWS_A)])
    pltpu.sync_copy(e2c_hbm, e2c_v)

    def body(k, carry):
        idx = ent_v[pl.ds(k * L, L)]
        g_v[pl.ds(k * L, L)] = plsc.load_gather(e2c_v, [idx])
        return carry

    lax.fori_loop(0, ROWS_A // L, body, 0)
    pltpu.sync_copy(g_v, g_hbm.at[pl.ds(base, ROWS_A)])

    zv = jnp.zeros((L,), jnp.float32)
    lane = lax.broadcasted_iota(jnp.int32, (L,), 0)

    def zrow(i, carry):
        for j in range(D // L):
            hist_v[i, pl.ds(j * L, L)] = zv
        return carry

    lax.fori_loop(0, HCR, zrow, 0)
    for k in range(HCR // L):
        rowidx_v[pl.ds(k * L, L)] = lane + (k * L)
    ntc = HCR // 8

    @pl.when(sid < ntc)
    def _zero_cnt():
        pltpu.sync_copy(hist_v.at[pl.ds(0, 8)],
                        cnt_sh.at[pl.ds(sid * 8, 8)])

    plsc.subcore_barrier()
    pltpu.sync_copy(att1_hbm, att1_v)
    pltpu.sync_copy(att2_hbm, att2_v)

    ones_v = jnp.ones((L,), jnp.float32)
    masks = [lane == j for j in range(L)]

    def chunk(c, carry):
        off = (c * NW + wid) * KA
        pltpu.sync_copy(dst_hbm.at[pl.ds(off, KA)], dstv)
        pltpu.sync_copy(et_hbm.at[pl.ds(off, KA)], etv)
        pltpu.sync_copy(nf_hbm.at[pl.ds(off, KA)], nfv)
        for k in range(KA // L):
            dv = dstv[pl.ds(k * L, L)]
            r = lax.shift_right_logical(dv, 7)
            q = lax.bitwise_and(dv, D - 1)
            for j in range(L):
                plsc.addupdate_scatter(hist_v, [r, q], ones_v, mask=masks[j])
            tv = etv[pl.ds(k * L, L)] * NB
            nv = nfv[pl.ds(k * L, L)]
            cidx = (lane + k * L) * NB
            for b in range(NB):
                a1 = plsc.load_gather(att1_v, [tv + b])
                plsc.store_scatter(cf1_v, [cidx + b], a1 * nv)
                a2 = plsc.load_gather(att2_v, [tv + b])
                plsc.store_scatter(cf2_v, [cidx + b], a2 * nv)
        pltpu.sync_copy(cf1_v, cf1_hbm.at[pl.ds(off * NB, KA * NB)])
        pltpu.sync_copy(cf2_v, cf2_hbm.at[pl.ds(off * NB, KA * NB)])
        return carry

    lax.fori_loop(0, NCH_A, chunk, 0)
    pltpu.sync_copy(hist_v, cnt_sh.at[rowidx_v], add=True)
    plsc.subcore_barrier()

    @pl.when(sid < ntc)
    def _write_cnt():
        pltpu.sync_copy(cnt_sh.at[pl.ds(sid * 8, 8)],
                        hist_v.at[pl.ds(0, 8)])
        pltpu.sync_copy(hist_v.at[pl.ds(0, 8)],
                        pc_hbm.at[pl.ds(cid * HCR + sid * 8, 8)])


STRIPE = NACC // NS


@functools.partial(
    pl.kernel,
    out_type=jax.ShapeDtypeStruct((2 * NACC, D), jnp.float32),
    mesh=_mesh,
    scratch_types=[
        pltpu.VMEM((KE,), jnp.int32),
        pltpu.VMEM((KE,), jnp.int32),
        pltpu.VMEM((KE,), jnp.int32),
        pltpu.VMEM((KE,), jnp.int32),
        pltpu.VMEM((KE * NB + L,), jnp.float32),
        pltpu.VMEM((KE * NB + L,), jnp.float32),
        pltpu.VMEM((KE, NB * D // 2), jnp.int32),
        pltpu.VMEM((KE, NB * D // 2), jnp.int32),
        pltpu.VMEM((KE, D), jnp.float32),
        pltpu.VMEM((KE, D), jnp.float32),
        pltpu.VMEM((KE,), jnp.int32),
        pltpu.VMEM((KE,), jnp.int32),
        pltpu.VMEM_SHARED((NACC, D), jnp.float32),
        pltpu.SemaphoreType.DMA,
        pltpu.SemaphoreType.DMA,
        pltpu.SemaphoreType.DMA,
        pltpu.SemaphoreType.DMA,
        pltpu.SemaphoreType.DMA,
        pltpu.SemaphoreType.DMA,
        pltpu.SemaphoreType.DMA,
        pltpu.SemaphoreType.DMA,
        pltpu.SemaphoreType.DMA,
        pltpu.SemaphoreType.DMA,
    ],
    compiler_params=_sc_params,
)
def _sc_edge(yb_hbm, src_hbm, dst_hbm, cf_hbm, z_hbm, p_hbm,
             srcv0, srcv1, dstv0, dstv1, cfv0, cfv1, rows0, rows1,
             msg0, msg1, dstm0, dstm1, acc_sh,
             ss0, ss1, sd0, sd1, sc0, sc1, sg0, sg1, sa0, sa1):
    cid = lax.axis_index("c")
    sid = lax.axis_index("s")
    wid = sid * NC + cid
    srcv = [srcv0, srcv1]
    dstv = [dstv0, dstv1]
    cfv = [cfv0, cfv1]
    rows = [rows0, rows1]
    msg = [msg0, msg1]
    dstm = [dstm0, dstm1]
    sem_s = [ss0, ss1]
    sem_d = [sd0, sd1]
    sem_c = [sc0, sc1]
    sem_g = [sg0, sg1]
    sem_a = [sa0, sa1]

    pltpu.sync_copy(z_hbm.at[pl.ds(sid * STRIPE, STRIPE)],
                    acc_sh.at[pl.ds(sid * STRIPE, STRIPE)])
    plsc.subcore_barrier()

    def issue_idx(g, s):
        off = (g * NW + wid) * KE
        pltpu.make_async_copy(src_hbm.at[pl.ds(off, KE)],
                              srcv[s], sem_s[s]).start()
        pltpu.make_async_copy(dst_hbm.at[pl.ds(off, KE)],
                              dstv[s], sem_d[s]).start()
        pltpu.make_async_copy(cf_hbm.at[pl.ds(off * NB, KE * NB)],
                              cfv[s].at[pl.ds(0, KE * NB)], sem_c[s]).start()

    def wait_idx(s):
        pltpu.make_async_copy(src_hbm.at[pl.ds(0, KE)],
                              srcv[s], sem_s[s]).wait()
        pltpu.make_async_copy(dst_hbm.at[pl.ds(0, KE)],
                              dstv[s], sem_d[s]).wait()
        pltpu.make_async_copy(cf_hbm.at[pl.ds(0, KE * NB)],
                              cfv[s].at[pl.ds(0, KE * NB)], sem_c[s]).wait()

    def start_gather(s):
        pltpu.make_async_copy(yb_hbm.at[srcv[s]], rows[s], sem_g[s]).start()

    def wait_gather(s):
        pltpu.make_async_copy(yb_hbm.at[srcv[s]], rows[s], sem_g[s]).wait()

    issue_idx(0, 0)
    wait_idx(0)
    start_gather(0)
    issue_idx(1, 1)

    def wait_scatter(s):
        pltpu.make_async_copy(msg[s], acc_sh.at[dstm[s]], sem_a[s]).wait()

    def half(g, cur, nxt):
        @pl.when(g + 1 < NCH_P)
        def _advance():
            wait_idx(nxt)
            start_gather(nxt)

        wait_gather(cur)

        @pl.when(g >= 2)
        def _drain():
            wait_scatter(cur)

        rv = rows[cur]
        cv_ref = cfv[cur]
        mv = msg[cur]

        def one_edge(e):
            cv = cv_ref[pl.ds(e * NB, L)]
            c = [cv[0], cv[1], cv[2], cv[3]]
            for j in range(D // (2 * L)):
                vlo = None
                for b in range(NB):
                    wv = rv[e, pl.ds((b * D // 2) + j * L, L)]
                    ab = plsc.bitcast(wv, jnp.bfloat16)
                    lo, hi = plsc.unpack(
                        ab, format=plsc.PackFormat.INTERLEAVED)
                    if vlo is None:
                        vlo = lo * c[b]
                        vhi = hi * c[b]
                    else:
                        vlo = vlo + lo * c[b]
                        vhi = vhi + hi * c[b]
                mv[e, pl.ds(j * 2 * L, L)] = vlo
                mv[e, pl.ds(j * 2 * L + L, L)] = vhi

        def edge2(i, ecarry):
            one_edge(2 * i)
            one_edge(2 * i + 1)
            return ecarry

        lax.fori_loop(0, KE // 2, edge2, 0)
        for k in range(KE // L):
            dstm[cur][pl.ds(k * L, L)] = dstv[cur][pl.ds(k * L, L)]
        pltpu.async_copy(mv, acc_sh.at[dstm[cur]], sem_a[cur], add=True)

        @pl.when(g + 2 < NCH_P)
        def _prefetch():
            issue_idx(g + 2, cur)

    def pair(p, carry):
        half(2 * p, 0, 1)
        half(2 * p + 1, 1, 0)
        return carry

    lax.fori_loop(0, NCH_P // 2, pair, 0)
    wait_scatter(0)
    wait_scatter(1)
    plsc.subcore_barrier()
    pltpu.sync_copy(acc_sh.at[pl.ds(sid * STRIPE, STRIPE)],
                    p_hbm.at[pl.ds(cid * NACC + sid * STRIPE, STRIPE)])


BM = 256
BN = 400
G = N // BN


def _mm_body(x_ref, w_ref, yb_ref, yr_ref):
    y = jnp.dot(x_ref[...], w_ref[...], preferred_element_type=jnp.float32)
    nw = NB * D // 2
    lo = lax.bitcast_convert_type(
        y[:, :nw].astype(jnp.bfloat16), jnp.uint16).astype(jnp.uint32)
    hi = lax.bitcast_convert_type(
        y[:, nw:2 * nw].astype(jnp.bfloat16), jnp.uint16).astype(jnp.uint32)
    word = jnp.bitwise_or(lax.shift_left(hi, jnp.uint32(16)), lo)
    yb_ref[...] = lax.bitcast_convert_type(word, jnp.int32)
    yr_ref[...] = y[:, 2 * nw:]


_mm = pl.pallas_call(
    _mm_body,
    grid=(NP // BM,),
    in_specs=[
        pl.BlockSpec((BM, D), lambda i: (i, 0)),
        pl.BlockSpec((D, (NB + 1) * D), lambda i: (0, 0)),
    ],
    out_specs=[
        pl.BlockSpec((BM, NB * D // 2), lambda i: (i, 0)),
        pl.BlockSpec((BM, D), lambda i: (i, 0)),
    ],
    out_shape=[
        jax.ShapeDtypeStruct((NP, NB * D // 2), jnp.int32),
        jax.ShapeDtypeStruct((NP, D), jnp.float32),
    ],
)


BE = 1024
PCB = BE // D


def _epi_body(p0_ref, p1_ref, pc0_ref, pc1_ref, yr_ref, b_ref, out_ref,
              *, relu):
    s = p0_ref[...] + p1_ref[...]
    pc = pc0_ref[...] + pc1_ref[...]
    oh = (lax.shift_right_logical(
              lax.broadcasted_iota(jnp.int32, (BE, PCB), 0), 7)
          == lax.broadcasted_iota(jnp.int32, (BE, PCB), 1)).astype(
              jnp.float32)
    psel = jnp.dot(oh, pc, preferred_element_type=jnp.float32)
    ri = lax.broadcasted_iota(jnp.int32, (BE, D), 0)
    li = lax.broadcasted_iota(jnp.int32, (BE, D), 1)
    cnt = jnp.sum(jnp.where(li == lax.bitwise_and(ri, D - 1), psel, 0.0),
                  axis=1, keepdims=True)
    o = s / jnp.maximum(cnt, 1.0) + yr_ref[...] + b_ref[...]
    if relu:
        o = jnp.maximum(o, 0.0)
    out_ref[...] = o


def _make_epi(relu):
    return pl.pallas_call(
        functools.partial(_epi_body, relu=relu),
        grid=(NP // BE,),
        in_specs=[
            pl.BlockSpec((BE, D), lambda i: (i, 0)),
            pl.BlockSpec((BE, D), lambda i: (NACC // BE + i, 0)),
            pl.BlockSpec((PCB, D), lambda i: (i, 0)),
            pl.BlockSpec((PCB, D), lambda i: (HCR // PCB + i, 0)),
            pl.BlockSpec((BE, D), lambda i: (i, 0)),
            pl.BlockSpec((D,), lambda i: (0,)),
        ],
        out_specs=pl.BlockSpec((BE, D), lambda i: (i, 0)),
        out_shape=jax.ShapeDtypeStruct((NP, D), jnp.float32),
    )


_epi_relu = _make_epi(True)
_epi_lin = _make_epi(False)


def _stats_body(x_ref, c_ref, cs_ref, cc_ref):
    @pl.when(pl.program_id(0) == 0)
    def _init():
        cs_ref[...] = jnp.zeros_like(cs_ref)
        cc_ref[...] = jnp.zeros_like(cc_ref)

    cls = c_ref[0, 0, :]
    oht = (lax.broadcasted_iota(jnp.int32, (CPAD, BN), 0)
           == cls[None, :]).astype(jnp.float32)
    cs_ref[...] += jnp.dot(oht, x_ref[...], preferred_element_type=jnp.float32)
    cc_ref[...] += jnp.broadcast_to(
        jnp.sum(oht, axis=1, keepdims=True), (CPAD, D))


_stats = pl.pallas_call(
    _stats_body,
    grid=(G,),
    in_specs=[
        pl.BlockSpec((BN, D), lambda i: (i, 0)),
        pl.BlockSpec((1, 1, BN), lambda i: (i, 0, 0)),
    ],
    out_specs=[
        pl.BlockSpec((CPAD, D), lambda i: (0, 0)),
        pl.BlockSpec((CPAD, D), lambda i: (0, 0)),
    ],
    out_shape=[
        jax.ShapeDtypeStruct((CPAD, D), jnp.float32),
        jax.ShapeDtypeStruct((CPAD, D), jnp.float32),
    ],
)


def _fin_body(e_ref, g_ref, cs_ref, cc_ref, w_ref, out_ref):
    fvs = jax.nn.sigmoid(cs_ref[...] / jnp.maximum(cc_ref[...], 1.0))
    gcls = g_ref[0, 0, :]
    oh = (gcls[:, None]
          == lax.broadcasted_iota(jnp.int32, (BN, CPAD), 1)).astype(jnp.float32)
    cf = jnp.dot(oh, fvs, preferred_element_type=jnp.float32)
    e2 = (jnp.dot(e_ref[...], w_ref[:D, :], preferred_element_type=jnp.float32)
          + jnp.dot(cf, w_ref[D:, :], preferred_element_type=jnp.float32))
    nrm = jnp.sqrt(jnp.sum(e2 * e2, axis=1, keepdims=True))
    out_ref[...] = e2 / jnp.maximum(nrm, 1e-12)


_fin = pl.pallas_call(
    _fin_body,
    grid=(G,),
    in_specs=[
        pl.BlockSpec((BN, D), lambda i: (i, 0)),
        pl.BlockSpec((1, 1, BN), lambda i: (i, 0, 0)),
        pl.BlockSpec((CPAD, D), lambda i: (0, 0)),
        pl.BlockSpec((CPAD, D), lambda i: (0, 0)),
        pl.BlockSpec((2 * D, D), lambda i: (0, 0)),
    ],
    out_specs=pl.BlockSpec((BN, D), lambda i: (i, 0)),
    out_shape=jax.ShapeDtypeStruct((N, D), jnp.float32),
)


def kernel(entity, edge_index, edge_type, edge_norm, emb_e,
           basis1, att1, root1, bias1, basis2, att2, root2, bias2,
           fc_w, entity2class):
    entity_p = jnp.concatenate(
        [entity, jnp.zeros((NP - N,), jnp.int32)])
    e2c_p = jnp.concatenate(
        [entity2class, jnp.zeros((NP - N,), jnp.int32)])
    npad = E_PAD - E
    src = jnp.concatenate([edge_index[0], jnp.zeros((npad,), jnp.int32)])
    dst = jnp.concatenate(
        [edge_index[1], jnp.full((npad,), NACC - 1, jnp.int32)])
    et_p = jnp.concatenate([edge_type, jnp.zeros((npad,), jnp.int32)])
    nf_p = jnp.concatenate([edge_norm, jnp.zeros((npad,), jnp.float32)])
    wb1 = jnp.concatenate(
        [basis1[0], basis1[1], basis1[2], basis1[3]], axis=1)
    wb2 = jnp.concatenate(
        [basis2[0], basis2[1], basis2[2], basis2[3]], axis=1)
    w1 = jnp.concatenate([wb1[:, _LO_IDX], wb1[:, _HI_IDX], root1], axis=1)
    w2 = jnp.concatenate([wb2[:, _LO_IDX], wb2[:, _HI_IDX], root2], axis=1)
    zeros_acc = jnp.zeros((NACC, D), jnp.float32)

    x0, g, pc, cf1, cf2 = _sc_gather(
        entity_p, emb_e, e2c_p, dst, et_p, nf_p,
        att1.reshape(-1), att2.reshape(-1))
    cs, cc = _stats(emb_e, entity2class.reshape(G, 1, BN))

    y1b, y1r = _mm(x0, w1)
    p1 = _sc_edge(y1b, src, dst, cf1, zeros_acc)
    e1 = _epi_relu(p1, p1, pc, pc, y1r, bias1)

    y2b, y2r = _mm(e1, w2)
    p2 = _sc_edge(y2b, src, dst, cf2, zeros_acc)
    e1f = _epi_lin(p2, p2, pc, pc, y2r, bias2)

    g3d = g[:N].reshape(G, 1, BN)
    return _fin(e1f, g3d, cs, cc, fc_w.T)


# speedup vs baseline: 1.2515x; 1.0012x over previous
import functools

import jax
import jax.numpy as jnp
import numpy as np
from jax import lax
from jax.experimental import pallas as pl
from jax.experimental.pallas import tpu as pltpu
from jax.experimental.pallas import tpu_sc as plsc

N = 10000
E = 160000
D = 128
R2 = 200
NB = 4
CPAD = 64

NC, NS, L = 2, 16, 16
NW = NC * NS
NP = 10240
NACC = 10240
HCR = NACC // D

_LO_IDX = np.array([32 * (k // 16) + k % 16 for k in range(256)])
_HI_IDX = _LO_IDX + 16

_mesh = plsc.VectorSubcoreMesh(
    core_axis_name="c", subcore_axis_name="s", num_cores=NC, num_subcores=NS)
_sc_params = pltpu.CompilerParams(needs_layout_passes=False)

ROWS_A = NP // NW

KE = 32
E_PAD = 163840
NCH_P = E_PAD // (NW * KE)
KA = 128
NCH_A = E_PAD // (NW * KA)


@functools.partial(
    pl.kernel,
    out_type=(jax.ShapeDtypeStruct((NP, D), jnp.float32),
              jax.ShapeDtypeStruct((NP,), jnp.int32),
              jax.ShapeDtypeStruct((2 * HCR, D), jnp.float32),
              jax.ShapeDtypeStruct((E_PAD * NB,), jnp.float32),
              jax.ShapeDtypeStruct((E_PAD * NB,), jnp.float32)),
    mesh=_mesh,
    scratch_types=[
        pltpu.VMEM((ROWS_A,), jnp.int32),
        pltpu.VMEM((ROWS_A, D), jnp.float32),
        pltpu.VMEM((NP,), jnp.int32),
        pltpu.VMEM((ROWS_A,), jnp.int32),
        pltpu.VMEM((KA,), jnp.int32),
        pltpu.VMEM((KA,), jnp.int32),
        pltpu.VMEM((KA,), jnp.float32),
        pltpu.VMEM((R2 * NB,), jnp.float32),
        pltpu.VMEM((R2 * NB,), jnp.float32),
        pltpu.VMEM((KA * NB,), jnp.float32),
        pltpu.VMEM((KA * NB,), jnp.float32),
        pltpu.VMEM((HCR, D), jnp.float32),
        pltpu.VMEM((HCR,), jnp.int32),
        pltpu.VMEM_SHARED((HCR, D), jnp.float32),
        pltpu.SemaphoreType.DMA,
    ],
    compiler_params=_sc_params,
)
def _sc_gather(ent_hbm, emb_hbm, e2c_hbm, dst_hbm, et_hbm, nf_hbm,
               att1_hbm, att2_hbm,
               x0_hbm, g_hbm, pc_hbm, cf1_hbm, cf2_hbm,
               ent_v, rows_v, e2c_v, g_v, dstv, etv, nfv,
               att1_v, att2_v, cf1_v, cf2_v, hist_v, rowidx_v, cnt_sh,
               sem):
    cid = lax.axis_index("c")
    sid = lax.axis_index("s")
    wid = sid * NC + cid
    base = wid * ROWS_A
    pltpu.sync_copy(ent_hbm.at[pl.ds(base, ROWS_A)], ent_v)
    pltpu.async_copy(emb_hbm.at[ent_v], rows_v, sem).wait()
    pltpu.sync_copy(rows_v, x0_hbm.at[pl.ds(base, ROWS_A)])
    pltpu.sync_copy(e2c_hbm, e2c_v)

    def body(k, carry):
        idx = ent_v[pl.ds(k * L, L)]
        g_v[pl.ds(k * L, L)] = plsc.load_gather(e2c_v, [idx])
        return carry

    lax.fori_loop(0, ROWS_A // L, body, 0)
    pltpu.sync_copy(g_v, g_hbm.at[pl.ds(base, ROWS_A)])

    zv = jnp.zeros((L,), jnp.float32)
    lane = lax.broadcasted_iota(jnp.int32, (L,), 0)

    def zrow(i, carry):
        for j in range(D // L):
            hist_v[i, pl.ds(j * L, L)] = zv
        return carry

    lax.fori_loop(0, HCR, zrow, 0)
    for k in range(HCR // L):
        rowidx_v[pl.ds(k * L, L)] = lane + (k * L)
    ntc = HCR // 8

    @pl.when(sid < ntc)
    def _zero_cnt():
        pltpu.sync_copy(hist_v.at[pl.ds(0, 8)],
                        cnt_sh.at[pl.ds(sid * 8, 8)])

    plsc.subcore_barrier()
    pltpu.sync_copy(att1_hbm, att1_v)
    pltpu.sync_copy(att2_hbm, att2_v)

    ones_v = jnp.ones((L,), jnp.float32)
    masks = [lane == j for j in range(L)]

    def chunk(c, carry):
        off = (c * NW + wid) * KA
        pltpu.sync_copy(dst_hbm.at[pl.ds(off, KA)], dstv)
        pltpu.sync_copy(et_hbm.at[pl.ds(off, KA)], etv)
        pltpu.sync_copy(nf_hbm.at[pl.ds(off, KA)], nfv)
        for k in range(KA // L):
            dv = dstv[pl.ds(k * L, L)]
            r = lax.shift_right_logical(dv, 7)
            q = lax.bitwise_and(dv, D - 1)
            for j in range(L):
                plsc.addupdate_scatter(hist_v, [r, q], ones_v, mask=masks[j])
            tv = etv[pl.ds(k * L, L)] * NB
            nv = nfv[pl.ds(k * L, L)]
            cidx = (lane + k * L) * NB
            for b in range(NB):
                a1 = plsc.load_gather(att1_v, [tv + b])
                plsc.store_scatter(cf1_v, [cidx + b], a1 * nv)
                a2 = plsc.load_gather(att2_v, [tv + b])
                plsc.store_scatter(cf2_v, [cidx + b], a2 * nv)
        pltpu.sync_copy(cf1_v, cf1_hbm.at[pl.ds(off * NB, KA * NB)])
        pltpu.sync_copy(cf2_v, cf2_hbm.at[pl.ds(off * NB, KA * NB)])
        return carry

    lax.fori_loop(0, NCH_A, chunk, 0)
    pltpu.sync_copy(hist_v, cnt_sh.at[rowidx_v], add=True)
    plsc.subcore_barrier()

    @pl.when(sid < ntc)
    def _write_cnt():
        pltpu.sync_copy(cnt_sh.at[pl.ds(sid * 8, 8)],
                        hist_v.at[pl.ds(0, 8)])
        pltpu.sync_copy(hist_v.at[pl.ds(0, 8)],
                        pc_hbm.at[pl.ds(cid * HCR + sid * 8, 8)])


STRIPE = NACC // NS


@functools.partial(
    pl.kernel,
    out_type=jax.ShapeDtypeStruct((2 * NACC, D), jnp.float32),
    mesh=_mesh,
    scratch_types=(
        [pltpu.VMEM((KE,), jnp.int32)] * 4
        + [pltpu.VMEM((KE,), jnp.int32)] * 4
        + [pltpu.VMEM((KE * NB + L,), jnp.float32)] * 4
        + [pltpu.VMEM((KE, NB * D // 2), jnp.int32)] * 4
        + [pltpu.VMEM((KE, D), jnp.float32)] * 2
        + [pltpu.VMEM((KE,), jnp.int32)] * 2
        + [pltpu.VMEM_SHARED((NACC, D), jnp.float32)]
        + [pltpu.SemaphoreType.DMA] * 18
    ),
    compiler_params=_sc_params,
)
def _sc_edge(yb_hbm, src_hbm, dst_hbm, cf_hbm, z_hbm, p_hbm,
             srcv0, srcv1, srcv2, srcv3, dstv0, dstv1, dstv2, dstv3,
             cfv0, cfv1, cfv2, cfv3, rows0, rows1, rows2, rows3,
             msg0, msg1, dstm0, dstm1, acc_sh,
             ss0, ss1, ss2, ss3, sd0, sd1, sd2, sd3,
             sc0, sc1, sc2, sc3, sg0, sg1, sg2, sg3, sa0, sa1):
    cid = lax.axis_index("c")
    sid = lax.axis_index("s")
    wid = sid * NC + cid
    srcv = [srcv0, srcv1, srcv2, srcv3]
    dstv = [dstv0, dstv1, dstv2, dstv3]
    cfv = [cfv0, cfv1, cfv2, cfv3]
    rows = [rows0, rows1, rows2, rows3]
    msg = [msg0, msg1]
    dstm = [dstm0, dstm1]
    sem_s = [ss0, ss1, ss2, ss3]
    sem_d = [sd0, sd1, sd2, sd3]
    sem_c = [sc0, sc1, sc2, sc3]
    sem_g = [sg0, sg1, sg2, sg3]
    sem_a = [sa0, sa1]

    pltpu.sync_copy(z_hbm.at[pl.ds(sid * STRIPE, STRIPE)],
                    acc_sh.at[pl.ds(sid * STRIPE, STRIPE)])
    plsc.subcore_barrier()

    def issue_idx(g, s):
        off = (g * NW + wid) * KE
        pltpu.make_async_copy(src_hbm.at[pl.ds(off, KE)],
                              srcv[s], sem_s[s]).start()
        pltpu.make_async_copy(dst_hbm.at[pl.ds(off, KE)],
                              dstv[s], sem_d[s]).start()
        pltpu.make_async_copy(cf_hbm.at[pl.ds(off * NB, KE * NB)],
                              cfv[s].at[pl.ds(0, KE * NB)], sem_c[s]).start()

    def wait_idx(s):
        pltpu.make_async_copy(src_hbm.at[pl.ds(0, KE)],
                              srcv[s], sem_s[s]).wait()
        pltpu.make_async_copy(dst_hbm.at[pl.ds(0, KE)],
                              dstv[s], sem_d[s]).wait()
        pltpu.make_async_copy(cf_hbm.at[pl.ds(0, KE * NB)],
                              cfv[s].at[pl.ds(0, KE * NB)], sem_c[s]).wait()

    def start_gather(s):
        pltpu.make_async_copy(yb_hbm.at[srcv[s]], rows[s], sem_g[s]).start()

    def wait_gather(s):
        pltpu.make_async_copy(yb_hbm.at[srcv[s]], rows[s], sem_g[s]).wait()

    issue_idx(0, 0)
    issue_idx(1, 1)
    issue_idx(2, 2)
    wait_idx(0)
    start_gather(0)
    wait_idx(1)
    start_gather(1)

    def wait_scatter(s):
        pltpu.make_async_copy(msg[s], acc_sh.at[dstm[s]], sem_a[s]).wait()

    def half(g, cur):
        @pl.when(g + 2 < NCH_P)
        def _advance():
            wait_idx((cur + 2) % 4)
            start_gather((cur + 2) % 4)

        wait_gather(cur)
        mcur = cur % 2

        @pl.when(g >= 2)
        def _drain():
            wait_scatter(mcur)

        rv = rows[cur]
        cv_ref = cfv[cur]
        mv = msg[mcur]

        def one_edge(e):
            cv = cv_ref[pl.ds(e * NB, L)]
            c = [cv[0], cv[1], cv[2], cv[3]]
            for j in range(D // (2 * L)):
                vlo = None
                for b in range(NB):
                    wv = rv[e, pl.ds((b * D // 2) + j * L, L)]
                    ab = plsc.bitcast(wv, jnp.bfloat16)
                    lo, hi = plsc.unpack(
                        ab, format=plsc.PackFormat.INTERLEAVED)
                    if vlo is None:
                        vlo = lo * c[b]
                        vhi = hi * c[b]
                    else:
                        vlo = vlo + lo * c[b]
                        vhi = vhi + hi * c[b]
                mv[e, pl.ds(j * 2 * L, L)] = vlo
                mv[e, pl.ds(j * 2 * L + L, L)] = vhi

        def edge2(i, ecarry):
            one_edge(2 * i)
            one_edge(2 * i + 1)
            return ecarry

        lax.fori_loop(0, KE // 2, edge2, 0)
        for k in range(KE // L):
            dstm[mcur][pl.ds(k * L, L)] = dstv[cur][pl.ds(k * L, L)]
        pltpu.async_copy(mv, acc_sh.at[dstm[mcur]], sem_a[mcur], add=True)

        @pl.when(g + 3 < NCH_P)
        def _prefetch():
            issue_idx(g + 3, (cur + 3) % 4)

    def quad(p, carry):
        half(4 * p, 0)
        half(4 * p + 1, 1)
        half(4 * p + 2, 2)
        half(4 * p + 3, 3)
        return carry

    lax.fori_loop(0, NCH_P // 4, quad, 0)
    wait_scatter(0)
    wait_scatter(1)
    plsc.subcore_barrier()
    pltpu.sync_copy(acc_sh.at[pl.ds(sid * STRIPE, STRIPE)],
                    p_hbm.at[pl.ds(cid * NACC + sid * STRIPE, STRIPE)])


BM = 256
BN = 400
G = N // BN


def _mm_body(x_ref, w_ref, yb_ref, yr_ref):
    y = jnp.dot(x_ref[...], w_ref[...], preferred_element_type=jnp.float32)
    nw = NB * D // 2
    lo = lax.bitcast_convert_type(
        y[:, :nw].astype(jnp.bfloat16), jnp.uint16).astype(jnp.uint32)
    hi = lax.bitcast_convert_type(
        y[:, nw:2 * nw].astype(jnp.bfloat16), jnp.uint16).astype(jnp.uint32)
    word = jnp.bitwise_or(lax.shift_left(hi, jnp.uint32(16)), lo)
    yb_ref[...] = lax.bitcast_convert_type(word, jnp.int32)
    yr_ref[...] = y[:, 2 * nw:]


_mm = pl.pallas_call(
    _mm_body,
    grid=(NP // BM,),
    in_specs=[
        pl.BlockSpec((BM, D), lambda i: (i, 0)),
        pl.BlockSpec((D, (NB + 1) * D), lambda i: (0, 0)),
    ],
    out_specs=[
        pl.BlockSpec((BM, NB * D // 2), lambda i: (i, 0)),
        pl.BlockSpec((BM, D), lambda i: (i, 0)),
    ],
    out_shape=[
        jax.ShapeDtypeStruct((NP, NB * D // 2), jnp.int32),
        jax.ShapeDtypeStruct((NP, D), jnp.float32),
    ],
)


BE = 1024
PCB = BE // D


def _epi_body(p0_ref, p1_ref, pc0_ref, pc1_ref, yr_ref, b_ref, out_ref,
              *, relu):
    s = p0_ref[...] + p1_ref[...]
    pc = pc0_ref[...] + pc1_ref[...]
    oh = (lax.shift_right_logical(
              lax.broadcasted_iota(jnp.int32, (BE, PCB), 0), 7)
          == lax.broadcasted_iota(jnp.int32, (BE, PCB), 1)).astype(
              jnp.float32)
    psel = jnp.dot(oh, pc, preferred_element_type=jnp.float32)
    ri = lax.broadcasted_iota(jnp.int32, (BE, D), 0)
    li = lax.broadcasted_iota(jnp.int32, (BE, D), 1)
    cnt = jnp.sum(jnp.where(li == lax.bitwise_and(ri, D - 1), psel, 0.0),
                  axis=1, keepdims=True)
    o = s / jnp.maximum(cnt, 1.0) + yr_ref[...] + b_ref[...]
    if relu:
        o = jnp.maximum(o, 0.0)
    out_ref[...] = o


def _make_epi(relu):
    return pl.pallas_call(
        functools.partial(_epi_body, relu=relu),
        grid=(NP // BE,),
        in_specs=[
            pl.BlockSpec((BE, D), lambda i: (i, 0)),
            pl.BlockSpec((BE, D), lambda i: (NACC // BE + i, 0)),
            pl.BlockSpec((PCB, D), lambda i: (i, 0)),
            pl.BlockSpec((PCB, D), lambda i: (HCR // PCB + i, 0)),
            pl.BlockSpec((BE, D), lambda i: (i, 0)),
            pl.BlockSpec((D,), lambda i: (0,)),
        ],
        out_specs=pl.BlockSpec((BE, D), lambda i: (i, 0)),
        out_shape=jax.ShapeDtypeStruct((NP, D), jnp.float32),
    )


_epi_relu = _make_epi(True)
_epi_lin = _make_epi(False)


def _stats_body(x_ref, c_ref, cs_ref, cc_ref):
    @pl.when(pl.program_id(0) == 0)
    def _init():
        cs_ref[...] = jnp.zeros_like(cs_ref)
        cc_ref[...] = jnp.zeros_like(cc_ref)

    cls = c_ref[0, 0, :]
    oht = (lax.broadcasted_iota(jnp.int32, (CPAD, BN), 0)
           == cls[None, :]).astype(jnp.float32)
    cs_ref[...] += jnp.dot(oht, x_ref[...], preferred_element_type=jnp.float32)
    cc_ref[...] += jnp.broadcast_to(
        jnp.sum(oht, axis=1, keepdims=True), (CPAD, D))


_stats = pl.pallas_call(
    _stats_body,
    grid=(G,),
    in_specs=[
        pl.BlockSpec((BN, D), lambda i: (i, 0)),
        pl.BlockSpec((1, 1, BN), lambda i: (i, 0, 0)),
    ],
    out_specs=[
        pl.BlockSpec((CPAD, D), lambda i: (0, 0)),
        pl.BlockSpec((CPAD, D), lambda i: (0, 0)),
    ],
    out_shape=[
        jax.ShapeDtypeStruct((CPAD, D), jnp.float32),
        jax.ShapeDtypeStruct((CPAD, D), jnp.float32),
    ],
)


def _fin_body(e_ref, g_ref, cs_ref, cc_ref, w_ref, out_ref):
    fvs = jax.nn.sigmoid(cs_ref[...] / jnp.maximum(cc_ref[...], 1.0))
    gcls = g_ref[0, 0, :]
    oh = (gcls[:, None]
          == lax.broadcasted_iota(jnp.int32, (BN, CPAD), 1)).astype(jnp.float32)
    cf = jnp.dot(oh, fvs, preferred_element_type=jnp.float32)
    e2 = (jnp.dot(e_ref[...], w_ref[:D, :], preferred_element_type=jnp.float32)
          + jnp.dot(cf, w_ref[D:, :], preferred_element_type=jnp.float32))
    nrm = jnp.sqrt(jnp.sum(e2 * e2, axis=1, keepdims=True))
    out_ref[...] = e2 / jnp.maximum(nrm, 1e-12)


_fin = pl.pallas_call(
    _fin_body,
    grid=(G,),
    in_specs=[
        pl.BlockSpec((BN, D), lambda i: (i, 0)),
        pl.BlockSpec((1, 1, BN), lambda i: (i, 0, 0)),
        pl.BlockSpec((CPAD, D), lambda i: (0, 0)),
        pl.BlockSpec((CPAD, D), lambda i: (0, 0)),
        pl.BlockSpec((2 * D, D), lambda i: (0, 0)),
    ],
    out_specs=pl.BlockSpec((BN, D), lambda i: (i, 0)),
    out_shape=jax.ShapeDtypeStruct((N, D), jnp.float32),
)


def kernel(entity, edge_index, edge_type, edge_norm, emb_e,
           basis1, att1, root1, bias1, basis2, att2, root2, bias2,
           fc_w, entity2class):
    entity_p = jnp.concatenate(
        [entity, jnp.zeros((NP - N,), jnp.int32)])
    e2c_p = jnp.concatenate(
        [entity2class, jnp.zeros((NP - N,), jnp.int32)])
    npad = E_PAD - E
    src = jnp.concatenate([edge_index[0], jnp.zeros((npad,), jnp.int32)])
    dst = jnp.concatenate(
        [edge_index[1], jnp.full((npad,), NACC - 1, jnp.int32)])
    et_p = jnp.concatenate([edge_type, jnp.zeros((npad,), jnp.int32)])
    nf_p = jnp.concatenate([edge_norm, jnp.zeros((npad,), jnp.float32)])
    wb1 = jnp.concatenate(
        [basis1[0], basis1[1], basis1[2], basis1[3]], axis=1)
    wb2 = jnp.concatenate(
        [basis2[0], basis2[1], basis2[2], basis2[3]], axis=1)
    w1 = jnp.concatenate([wb1[:, _LO_IDX], wb1[:, _HI_IDX], root1], axis=1)
    w2 = jnp.concatenate([wb2[:, _LO_IDX], wb2[:, _HI_IDX], root2], axis=1)
    zeros_acc = jnp.zeros((NACC, D), jnp.float32)

    x0, g, pc, cf1, cf2 = _sc_gather(
        entity_p, emb_e, e2c_p, dst, et_p, nf_p,
        att1.reshape(-1), att2.reshape(-1))
    cs, cc = _stats(emb_e, entity2class.reshape(G, 1, BN))

    y1b, y1r = _mm(x0, w1)
    p1 = _sc_edge(y1b, src, dst, cf1, zeros_acc)
    e1 = _epi_relu(p1, p1, pc, pc, y1r, bias1)

    y2b, y2r = _mm(e1, w2)
    p2 = _sc_edge(y2b, src, dst, cf2, zeros_acc)
    e1f = _epi_lin(p2, p2, pc, pc, y2r, bias2)

    g3d = g[:N].reshape(G, 1, BN)
    return _fin(e1f, g3d, cs, cc, fc_w.T)
